# baseline (speedup 1.0000x reference)
import jax
import jax.numpy as jnp
from jax import lax
from jax.experimental import pallas as pl
from jax.experimental.pallas import tpu as pltpu
from jax.experimental.pallas import tpu_sc as plsc

D_MODEL = 4096
N_EXP = 64
TOPK = 8
BT = 2048
NK = 2
DK = D_MODEL // NK

NC, NS, L = 2, 16, 16
NW = NC * NS


def _gate_body(x_ref, w_ref, gs_ref):
    k = pl.program_id(1)
    part = jax.lax.dot_general(
        x_ref[...], w_ref[...], (((1,), (1,)), ((), ())),
        preferred_element_type=jnp.float32,
    )

    @pl.when(k == 0)
    def _():
        gs_ref[...] = part

    @pl.when(k == NK - 1)
    def _():
        logits = gs_ref[...] + part if NK > 1 else part
        m = jnp.max(logits, axis=-1, keepdims=True)
        e = jnp.exp(logits - m)
        gs_ref[...] = e / jnp.sum(e, axis=-1, keepdims=True)

    if NK > 2:

        @pl.when((k > 0) & (k < NK - 1))
        def _():
            gs_ref[...] = gs_ref[...] + part


def _gate_softmax(x, W):
    B = x.shape[0]
    return pl.pallas_call(
        _gate_body,
        grid=(B // BT, NK),
        in_specs=[
            pl.BlockSpec((BT, DK), lambda i, k: (i, k)),
            pl.BlockSpec((N_EXP, DK), lambda i, k: (0, k)),
        ],
        out_specs=pl.BlockSpec((BT, N_EXP), lambda i, k: (i, 0)),
        out_shape=jax.ShapeDtypeStruct((B, N_EXP), jnp.float32),
    )(x, W)


def _make_sc_topk(B):
    rows_w = B // NW
    mesh = plsc.VectorSubcoreMesh(
        core_axis_name="c", subcore_axis_name="s", num_cores=NC, num_subcores=NS
    )

    def body(gs_hbm, oi_hbm, ow_hbm, gs_v, oi_v, ow_v):
        wid = lax.axis_index("s") * NC + lax.axis_index("c")
        base = wid * rows_w
        pltpu.sync_copy(gs_hbm.at[pl.ds(base * N_EXP, rows_w * N_EXP)], gs_v)
        lane = lax.iota(jnp.int32, L)
        m8 = lane < TOPK

        rev8 = jnp.where(m8, TOPK - 1 - lane, lane)
        shl8 = jnp.where(m8, lane, lane - TOPK)

        def gat(x, idx):
            return x.at[idx].get(mode="promise_in_bounds")

        def halfclean8(ka, va, kb, vb):
            kbp, vbp = gat(kb, rev8), gat(vb, rev8)
            take_a = ka >= kbp
            return jnp.where(take_a, ka, kbp), jnp.where(take_a, va, vbp)

        def one_row(r):
            basep = r * N_EXP
            sk, sv = [], []
            for c in range(N_EXP // L):
                key = gs_v[pl.ds(basep + c * L, L)]
                k, v = plsc.sort_key_val(key, lane + c * L, descending=True)
                sk.append(k)
                sv.append(v)
            k01, v01 = halfclean8(sk[0], sv[0], sk[1], sv[1])
            k23, v23 = halfclean8(sk[2], sv[2], sk[3], sv[3])
            kc = jnp.where(m8, k01, gat(k23, shl8))
            vc = jnp.where(m8, v01, gat(v23, shl8))
            kf, vf = plsc.sort_key_val(kc, vc, descending=True)
            cs = plsc.cumsum(jnp.where(m8, kf, 0.0))
            total = cs.at[jnp.full((L,), L - 1, jnp.int32)].get(
                mode="promise_in_bounds"
            )
            pos = r * TOPK + lane
            plsc.store_scatter(ow_v, [pos], kf / (total + 1e-8), mask=m8)
            plsc.store_scatter(oi_v, [pos], vf, mask=m8)

        @plsc.parallel_loop(0, rows_w, 1, unroll=4)
        def _rows(r):
            one_row(r)
        pltpu.sync_copy(oi_v, oi_hbm.at[pl.ds(base * TOPK, rows_w * TOPK)])
        pltpu.sync_copy(ow_v, ow_hbm.at[pl.ds(base * TOPK, rows_w * TOPK)])

    return pl.kernel(
        body,
        out_type=[
            jax.ShapeDtypeStruct((B * TOPK,), jnp.int32),
            jax.ShapeDtypeStruct((B * TOPK,), jnp.float32),
        ],
        mesh=mesh,
        compiler_params=pltpu.CompilerParams(needs_layout_passes=False),
        scratch_types=[
            pltpu.VMEM((rows_w * N_EXP,), jnp.float32),
            pltpu.VMEM((rows_w * TOPK,), jnp.int32),
            pltpu.VMEM((rows_w * TOPK,), jnp.float32),
        ],
    )


def kernel(x, W):
    if x.ndim == 3:
        x = x.mean(axis=1)
    B = x.shape[0]
    gs = _gate_softmax(x, W)
    top_idx, top_w = _make_sc_topk(B)(gs.reshape(-1))
    return (gs, top_idx.reshape(B, TOPK), top_w.reshape(B, TOPK))


# speedup vs baseline: 1.0275x; 1.0275x over previous
import jax
import jax.numpy as jnp
from jax import lax
from jax.experimental import pallas as pl
from jax.experimental.pallas import tpu as pltpu
from jax.experimental.pallas import tpu_sc as plsc

D_MODEL = 4096
N_EXP = 64
TOPK = 8
BT = 1024
NK = 1
DK = D_MODEL // NK

NC, NS, L = 2, 16, 16
NW = NC * NS


def _gate_body(x_ref, w_ref, gs_ref):
    k = pl.program_id(1)
    part = jax.lax.dot_general(
        x_ref[...], w_ref[...], (((1,), (1,)), ((), ())),
        preferred_element_type=jnp.float32,
    )

    @pl.when(k == 0)
    def _():
        gs_ref[...] = part

    @pl.when(k == NK - 1)
    def _():
        logits = gs_ref[...] + part if NK > 1 else part
        m = jnp.max(logits, axis=-1, keepdims=True)
        e = jnp.exp(logits - m)
        gs_ref[...] = e / jnp.sum(e, axis=-1, keepdims=True)

    if NK > 2:

        @pl.when((k > 0) & (k < NK - 1))
        def _():
            gs_ref[...] = gs_ref[...] + part


def _gate_softmax(x, W):
    B = x.shape[0]
    return pl.pallas_call(
        _gate_body,
        grid=(B // BT, NK),
        in_specs=[
            pl.BlockSpec((BT, DK), lambda i, k: (i, k)),
            pl.BlockSpec((N_EXP, DK), lambda i, k: (0, k)),
        ],
        out_specs=pl.BlockSpec((BT, N_EXP), lambda i, k: (i, 0)),
        out_shape=jax.ShapeDtypeStruct((B, N_EXP), jnp.float32),
    )(x, W)


def _make_sc_topk(B):
    rows_w = B // NW
    mesh = plsc.VectorSubcoreMesh(
        core_axis_name="c", subcore_axis_name="s", num_cores=NC, num_subcores=NS
    )

    def body(gs_hbm, oi_hbm, ow_hbm, gs_v, oi_v, ow_v):
        wid = lax.axis_index("s") * NC + lax.axis_index("c")
        base = wid * rows_w
        pltpu.sync_copy(gs_hbm.at[pl.ds(base * N_EXP, rows_w * N_EXP)], gs_v)
        lane = lax.iota(jnp.int32, L)
        m8 = lane < TOPK

        rev8 = jnp.where(m8, TOPK - 1 - lane, lane)
        shl8 = jnp.where(m8, lane, lane - TOPK)

        def gat(x, idx):
            return x.at[idx].get(mode="promise_in_bounds")

        def halfclean8(ka, va, kb, vb):
            kbp, vbp = gat(kb, rev8), gat(vb, rev8)
            take_a = ka >= kbp
            return jnp.where(take_a, ka, kbp), jnp.where(take_a, va, vbp)

        def one_row(r):
            basep = r * N_EXP
            sk, sv = [], []
            for c in range(N_EXP // L):
                key = gs_v[pl.ds(basep + c * L, L)]
                k, v = plsc.sort_key_val(key, lane + c * L, descending=True)
                sk.append(k)
                sv.append(v)
            k01, v01 = halfclean8(sk[0], sv[0], sk[1], sv[1])
            k2p, v2p = gat(sk[2], shl8), gat(sv[2], shl8)
            k3p, v3p = lax.rev(sk[3], (0,)), lax.rev(sv[3], (0,))
            take2 = k2p >= k3p
            kc = jnp.where(m8, k01, jnp.where(take2, k2p, k3p))
            vc = jnp.where(m8, v01, jnp.where(take2, v2p, v3p))
            kf, vf = plsc.sort_key_val(kc, vc, descending=True)
            cs = plsc.cumsum(jnp.where(m8, kf, 0.0))
            total = cs.at[jnp.full((L,), L - 1, jnp.int32)].get(
                mode="promise_in_bounds"
            )
            pos = r * TOPK + lane
            plsc.store_scatter(ow_v, [pos], kf / (total + 1e-8), mask=m8)
            plsc.store_scatter(oi_v, [pos], vf, mask=m8)

        @plsc.parallel_loop(0, rows_w, 1, unroll=4)
        def _rows(r):
            one_row(r)
        pltpu.sync_copy(oi_v, oi_hbm.at[pl.ds(base * TOPK, rows_w * TOPK)])
        pltpu.sync_copy(ow_v, ow_hbm.at[pl.ds(base * TOPK, rows_w * TOPK)])

    return pl.kernel(
        body,
        out_type=[
            jax.ShapeDtypeStruct((B * TOPK,), jnp.int32),
            jax.ShapeDtypeStruct((B * TOPK,), jnp.float32),
        ],
        mesh=mesh,
        compiler_params=pltpu.CompilerParams(needs_layout_passes=False),
        scratch_types=[
            pltpu.VMEM((rows_w * N_EXP,), jnp.float32),
            pltpu.VMEM((rows_w * TOPK,), jnp.int32),
            pltpu.VMEM((rows_w * TOPK,), jnp.float32),
        ],
    )


def kernel(x, W):
    if x.ndim == 3:
        x = x.mean(axis=1)
    B = x.shape[0]
    gs = _gate_softmax(x, W)
    top_idx, top_w = _make_sc_topk(B)(gs.reshape(-1))
    return (gs, top_idx.reshape(B, TOPK), top_w.reshape(B, TOPK))


# speedup vs baseline: 1.0290x; 1.0015x over previous
import jax
import jax.numpy as jnp
from jax import lax
from jax.experimental import pallas as pl
from jax.experimental.pallas import tpu as pltpu
from jax.experimental.pallas import tpu_sc as plsc

D_MODEL = 4096
N_EXP = 64
TOPK = 8
BT = 1024
NK = 1
DK = D_MODEL // NK

NC, NS, L = 2, 16, 16
NW = NC * NS


def _gate_body(x_ref, w_ref, gs_ref):
    k = pl.program_id(1)
    part = jax.lax.dot_general(
        x_ref[...], w_ref[...], (((1,), (1,)), ((), ())),
        preferred_element_type=jnp.float32,
    )

    @pl.when(k == 0)
    def _():
        gs_ref[...] = part

    @pl.when(k == NK - 1)
    def _():
        logits = gs_ref[...] + part if NK > 1 else part
        m = jnp.max(logits, axis=-1, keepdims=True)
        e = jnp.exp(logits - m)
        gs_ref[...] = e / jnp.sum(e, axis=-1, keepdims=True)

    if NK > 2:

        @pl.when((k > 0) & (k < NK - 1))
        def _():
            gs_ref[...] = gs_ref[...] + part


def _gate_softmax(x, W):
    B = x.shape[0]
    return pl.pallas_call(
        _gate_body,
        grid=(B // BT, NK),
        in_specs=[
            pl.BlockSpec((BT, DK), lambda i, k: (i, k)),
            pl.BlockSpec((N_EXP, DK), lambda i, k: (0, k)),
        ],
        out_specs=pl.BlockSpec((BT, N_EXP), lambda i, k: (i, 0)),
        out_shape=jax.ShapeDtypeStruct((B, N_EXP), jnp.float32),
    )(x, W)


def _make_sc_topk(B):
    rows_w = B // NW
    mesh = plsc.VectorSubcoreMesh(
        core_axis_name="c", subcore_axis_name="s", num_cores=NC, num_subcores=NS
    )

    n_chunks = 4
    rows_c = rows_w // n_chunks

    def body(gs_hbm, oi_hbm, ow_hbm, b0, b1, oi_v, ow_v, sem0, sem1):
        wid = lax.axis_index("s") * NC + lax.axis_index("c")
        base = wid * rows_w
        bufs = (b0, b1)
        sems = (sem0, sem1)

        def start(c):
            return pltpu.async_copy(
                gs_hbm.at[
                    pl.ds((base + c * rows_c) * N_EXP, rows_c * N_EXP)
                ],
                bufs[c % 2],
                sems[c % 2],
            )

        lane = lax.iota(jnp.int32, L)
        m8 = lane < TOPK

        rev8 = jnp.where(m8, TOPK - 1 - lane, lane)
        shl8 = jnp.where(m8, lane, lane - TOPK)

        def gat(x, idx):
            return x.at[idx].get(mode="promise_in_bounds")

        def halfclean8(ka, va, kb, vb):
            kbp, vbp = gat(kb, rev8), gat(vb, rev8)
            take_a = ka >= kbp
            return jnp.where(take_a, ka, kbp), jnp.where(take_a, va, vbp)

        def one_row(buf, r, rg):
            basep = r * N_EXP
            sk, sv = [], []
            for c in range(N_EXP // L):
                key = buf[pl.ds(basep + c * L, L)]
                k, v = plsc.sort_key_val(key, lane + c * L, descending=True)
                sk.append(k)
                sv.append(v)
            k01, v01 = halfclean8(sk[0], sv[0], sk[1], sv[1])
            k2p, v2p = gat(sk[2], shl8), gat(sv[2], shl8)
            k3p, v3p = lax.rev(sk[3], (0,)), lax.rev(sv[3], (0,))
            take2 = k2p >= k3p
            kc = jnp.where(m8, k01, jnp.where(take2, k2p, k3p))
            vc = jnp.where(m8, v01, jnp.where(take2, v2p, v3p))
            kf, vf = plsc.sort_key_val(kc, vc, descending=True)
            cs = plsc.cumsum(jnp.where(m8, kf, 0.0))
            total = cs.at[jnp.full((L,), L - 1, jnp.int32)].get(
                mode="promise_in_bounds"
            )
            pos = rg * TOPK + lane
            plsc.store_scatter(ow_v, [pos], kf / (total + 1e-8), mask=m8)
            plsc.store_scatter(oi_v, [pos], vf, mask=m8)

        cps = [start(0)]
        for c in range(n_chunks):
            cps[c].wait()
            if c + 1 < n_chunks:
                cps.append(start(c + 1))

            @plsc.parallel_loop(0, rows_c, 1, unroll=4)
            def _rows(r, buf=bufs[c % 2], roff=c * rows_c):
                one_row(buf, r, roff + r)

        pltpu.sync_copy(oi_v, oi_hbm.at[pl.ds(base * TOPK, rows_w * TOPK)])
        pltpu.sync_copy(ow_v, ow_hbm.at[pl.ds(base * TOPK, rows_w * TOPK)])

    return pl.kernel(
        body,
        out_type=[
            jax.ShapeDtypeStruct((B * TOPK,), jnp.int32),
            jax.ShapeDtypeStruct((B * TOPK,), jnp.float32),
        ],
        mesh=mesh,
        compiler_params=pltpu.CompilerParams(needs_layout_passes=False),
        scratch_types=[
            pltpu.VMEM((rows_w // n_chunks * N_EXP,), jnp.float32),
            pltpu.VMEM((rows_w // n_chunks * N_EXP,), jnp.float32),
            pltpu.VMEM((rows_w * TOPK,), jnp.int32),
            pltpu.VMEM((rows_w * TOPK,), jnp.float32),
            pltpu.SemaphoreType.DMA,
            pltpu.SemaphoreType.DMA,
        ],
    )


def kernel(x, W):
    if x.ndim == 3:
        x = x.mean(axis=1)
    B = x.shape[0]
    gs = _gate_softmax(x, W)
    top_idx, top_w = _make_sc_topk(B)(gs.reshape(-1))
    return (gs, top_idx.reshape(B, TOPK), top_w.reshape(B, TOPK))
